# baseline (speedup 1.0000x reference)
import functools

import jax
import jax.numpy as jnp
from jax import lax
from jax.experimental import pallas as pl
from jax.experimental.pallas import tpu as pltpu
from jax.experimental.pallas import tpu_sc as plsc

NUM_ITEM = 1000000
HIDDEN = 64
SEQ = 200
BATCH = 4096
PAD_IDX = 3

NC = 2
NS = 16
LANES = 16
NW = NC * NS
N = BATCH * SEQ
ROWS_PW = N // NW
GSZ = 128
BT = BATCH // GSZ
NG = ROWS_PW // GSZ
PADH = 2 * HIDDEN
OST = GSZ + 1
NBUF = 4
IBUF = 8
RUNROLL = 4


def _emb_body(xf, table, posf, out3, mask_out, pos_v,
              gbufs, obufs, ibufs, mask_bs, gsems, osems, isems):
    wid = lax.axis_index("s") * NC + lax.axis_index("c")
    base = wid * ROWS_PW
    g0 = wid * NG
    pltpu.sync_copy(posf, pos_v)

    def idx_copy(l, p):
        return pltpu.make_async_copy(
            xf.at[pl.ds(base + l * GSZ, GSZ)], ibufs[p], isems[p])

    def gather_copy(p, q):
        return pltpu.make_async_copy(
            table.at[ibufs[p]], gbufs[q], gsems[q])

    def out_copy(l, obuf, mbuf, sem):
        g = g0 + l
        s = g // BT
        bt = g - s * BT
        return (
            pltpu.make_async_copy(
                obuf.at[:, pl.ds(0, GSZ)],
                out3.at[s, :, pl.ds(bt * GSZ, GSZ)], sem),
            pltpu.make_async_copy(
                mbuf, mask_out.at[pl.ds(base + l * GSZ, GSZ)], sem),
        )

    iota16 = lax.iota(jnp.int32, LANES)
    hvecs = [iota16 + j * LANES for j in range(HIDDEN // LANES)]
    for p in range(IBUF):
        idx_copy(p, p).start()
    for q in range(NBUF):
        idx_copy(q, q).wait()
        gather_copy(q, q).start()

    def oct_body(qq, carry):
        for q8 in range(IBUF):
            l = qq * IBUF + q8
            q = q8 % NBUF
            gbuf, obuf, mbuf = gbufs[q], obufs[q8 % 2], mask_bs[q8 % 2]
            @pl.when(l >= 2)
            def _():
                for h in out_copy(l - 2, obuf, mbuf, osems[q8 % 2]):
                    h.wait()
            ibuf = ibufs[q8]
            npvs = []
            for i in range(GSZ // LANES):
                v = ibuf[pl.ds(i * LANES, LANES)]
                ispad = v == PAD_IDX
                mbuf[pl.ds(i * LANES, LANES)] = jnp.where(ispad, 1, 0)
                npvs.append(jnp.where(ispad, 0.0, 1.0))
            gather_copy(q8, q).wait()
            s = (g0 + l) // BT
            pvs = [pos_v[pl.ds(s * HIDDEN + j * LANES, LANES)]
                   for j in range(HIDDEN // LANES)]

            def row_body(r0, carry2):
                for u in range(RUNROLL):
                    r = r0 * RUNROLL + u
                    rvec = jnp.broadcast_to(r, (LANES,))
                    for j in range(HIDDEN // LANES):
                        gv = gbuf[r, pl.ds(j * LANES, LANES)]
                        plsc.store_scatter(obuf, [hvecs[j], rvec],
                                           gv + pvs[j])
                return carry2

            lax.fori_loop(0, GSZ // RUNROLL, row_body, 0, unroll=False)

            def mask_body(h, carry3):
                for i in range(GSZ // LANES):
                    ov = obuf[h, pl.ds(i * LANES, LANES)]
                    obuf[h, pl.ds(i * LANES, LANES)] = ov * npvs[i]
                return carry3

            lax.fori_loop(0, HIDDEN, mask_body, 0, unroll=False)
            for h in out_copy(l, obuf, mbuf, osems[q8 % 2]):
                h.start()
            @pl.when(l + NBUF < NG)
            def _():
                idx_copy(l + NBUF, (q8 + NBUF) % IBUF).wait()
                gather_copy((q8 + NBUF) % IBUF, q).start()
            @pl.when(l + IBUF < NG)
            def _():
                idx_copy(l + IBUF, q8).start()
        return carry

    lax.fori_loop(0, NG // IBUF, oct_body, 0, unroll=False)
    for h in out_copy(NG - 2, obufs[0], mask_bs[0], osems[0]):
        h.wait()
    for h in out_copy(NG - 1, obufs[1], mask_bs[1], osems[1]):
        h.wait()


_emb_call = pl.kernel(
    _emb_body,
    out_type=[
        jax.ShapeDtypeStruct((SEQ, HIDDEN, BATCH), jnp.float32),
        jax.ShapeDtypeStruct((N,), jnp.int32),
    ],
    mesh=plsc.VectorSubcoreMesh(
        core_axis_name="c", subcore_axis_name="s", num_cores=NC,
        num_subcores=NS),
    scratch_types=[
        pltpu.VMEM((SEQ * HIDDEN + LANES,), jnp.float32),
        [pltpu.VMEM((GSZ, PADH), jnp.float32)] * NBUF,
        [pltpu.VMEM((HIDDEN, OST), jnp.float32)] * 2,
        [pltpu.VMEM((GSZ,), jnp.int32)] * IBUF,
        [pltpu.VMEM((GSZ,), jnp.int32)] * 2,
        [pltpu.SemaphoreType.DMA] * NBUF,
        [pltpu.SemaphoreType.DMA] * 2,
        [pltpu.SemaphoreType.DMA] * IBUF,
    ],
    compiler_params=pltpu.CompilerParams(needs_layout_passes=False),
)


def kernel(x, item_table, pos_table):
    xf = x.T.reshape(N)
    tp = jnp.pad(item_table, ((0, 0), (0, PADH - HIDDEN)))
    posf = jnp.pad(pos_table.reshape(SEQ * HIDDEN), (0, LANES))
    out3, mask_i32 = _emb_call(xf, tp, posf)
    input_emb = out3.transpose(2, 0, 1)
    pad_masking = mask_i32.reshape(SEQ, BATCH).T.astype(bool)
    return (input_emb, pad_masking)


# speedup vs baseline: 1.6057x; 1.6057x over previous
import functools

import jax
import jax.numpy as jnp
from jax import lax
from jax.experimental import pallas as pl
from jax.experimental.pallas import tpu as pltpu
from jax.experimental.pallas import tpu_sc as plsc

NUM_ITEM = 1000000
HIDDEN = 64
SEQ = 200
BATCH = 4096
PAD_IDX = 3

NC = 2
NS = 16
LANES = 16
NW = NC * NS
N = BATCH * SEQ
ROWS_PW = N // NW
GSZ = 128
BT = BATCH // GSZ
NG = ROWS_PW // GSZ
PADH = 2 * HIDDEN
HT = HIDDEN // 8
OST = GSZ + 1
NBUF = 4
IBUF = 8
RUNROLL = 4
PAD2 = 2 * PAD_IDX


def _emb_body(xf, table, posf, out5, mask_out, pos_v,
              gbufs, obufs, ibufs, mask_bs, gsems, osems, isems):
    wid = lax.axis_index("s") * NC + lax.axis_index("c")
    base = wid * ROWS_PW
    g0 = wid * NG
    pltpu.sync_copy(posf, pos_v)

    def idx_copy(l, p):
        return pltpu.make_async_copy(
            xf.at[pl.ds(base + l * GSZ, GSZ)], ibufs[p], isems[p])

    def gather_copy(p, q):
        return pltpu.make_async_copy(
            table.at[ibufs[p]], gbufs[q], gsems[q])

    def out_copy(l, obuf, mbuf, sem):
        g = g0 + l
        s = g // BT
        bt = g - s * BT
        return (
            pltpu.make_async_copy(
                obuf.at[:, :, pl.ds(0, GSZ)],
                out5.at[s, :, bt, :, :], sem),
            pltpu.make_async_copy(
                mbuf, mask_out.at[pl.ds(base + l * GSZ, GSZ)], sem),
        )

    iota16 = lax.iota(jnp.int32, LANES)
    hvecs = [iota16 + j * LANES for j in range(HIDDEN // LANES)]
    htv = [hv // 8 for hv in hvecs]
    hmv = [hv % 8 for hv in hvecs]
    for p in range(IBUF):
        idx_copy(p, p).start()
    for q in range(NBUF):
        idx_copy(q, q).wait()
        gather_copy(q, q).start()

    def oct_body(qq, carry):
        for q8 in range(IBUF):
            l = qq * IBUF + q8
            q = q8 % NBUF
            gbuf, obuf, mbuf = gbufs[q], obufs[q8 % 2], mask_bs[q8 % 2]
            @pl.when(l >= 2)
            def _():
                for h in out_copy(l - 2, obuf, mbuf, osems[q8 % 2]):
                    h.wait()
            ibuf = ibufs[q8]
            npvs = []
            for i in range(GSZ // LANES):
                v = ibuf[pl.ds(i * LANES, LANES)]
                ispad = v == PAD2
                mbuf[pl.ds(i * LANES, LANES)] = jnp.where(ispad, 1, 0)
                npvs.append(jnp.where(ispad, 0.0, 1.0))
            gather_copy(q8, q).wait()
            s = (g0 + l) // BT
            pvs = [pos_v[pl.ds(s * HIDDEN + j * LANES, LANES)]
                   for j in range(HIDDEN // LANES)]

            def row_body(r0, carry2):
                for u in range(RUNROLL):
                    r = r0 * RUNROLL + u
                    rvec = jnp.broadcast_to(r, (LANES,))
                    for j in range(HIDDEN // LANES):
                        gv = gbuf[r, pl.ds(j * LANES, LANES)]
                        plsc.store_scatter(obuf, [htv[j], hmv[j], rvec],
                                           gv + pvs[j])
                return carry2

            lax.fori_loop(0, GSZ // RUNROLL, row_body, 0, unroll=False)

            def mask_body(h, carry3):
                for i in range(GSZ // LANES):
                    ov = obuf[h // 8, h % 8, pl.ds(i * LANES, LANES)]
                    obuf[h // 8, h % 8, pl.ds(i * LANES, LANES)] = (
                        ov * npvs[i])
                return carry3

            lax.fori_loop(0, HIDDEN, mask_body, 0, unroll=False)
            for h in out_copy(l, obuf, mbuf, osems[q8 % 2]):
                h.start()
            @pl.when(l + NBUF < NG)
            def _():
                idx_copy(l + NBUF, (q8 + NBUF) % IBUF).wait()
                gather_copy((q8 + NBUF) % IBUF, q).start()
            @pl.when(l + IBUF < NG)
            def _():
                idx_copy(l + IBUF, q8).start()
        return carry

    lax.fori_loop(0, NG // IBUF, oct_body, 0, unroll=False)
    for h in out_copy(NG - 2, obufs[0], mask_bs[0], osems[0]):
        h.wait()
    for h in out_copy(NG - 1, obufs[1], mask_bs[1], osems[1]):
        h.wait()


_emb_call = pl.kernel(
    _emb_body,
    out_type=[
        jax.ShapeDtypeStruct((SEQ, HT, BT, 8, GSZ), jnp.float32),
        jax.ShapeDtypeStruct((N,), jnp.int32),
    ],
    mesh=plsc.VectorSubcoreMesh(
        core_axis_name="c", subcore_axis_name="s", num_cores=NC,
        num_subcores=NS),
    scratch_types=[
        pltpu.VMEM((SEQ * HIDDEN + LANES,), jnp.float32),
        [pltpu.VMEM((GSZ, HIDDEN), jnp.float32)] * NBUF,
        [pltpu.VMEM((HT, 8, OST), jnp.float32)] * 2,
        [pltpu.VMEM((GSZ,), jnp.int32)] * IBUF,
        [pltpu.VMEM((GSZ,), jnp.int32)] * 2,
        [pltpu.SemaphoreType.DMA] * NBUF,
        [pltpu.SemaphoreType.DMA] * 2,
        [pltpu.SemaphoreType.DMA] * IBUF,
    ],
    compiler_params=pltpu.CompilerParams(
        needs_layout_passes=False, use_tc_tiling_on_sc=False),
)


def kernel(x, item_table, pos_table):
    xf = x.T.reshape(N) * 2
    tp = jnp.pad(item_table, ((0, 0), (0, PADH - HIDDEN)))
    tp2 = tp.reshape(2 * NUM_ITEM, HIDDEN)
    posf = jnp.pad(pos_table.reshape(SEQ * HIDDEN), (0, LANES))
    out5, mask_i32 = _emb_call(xf, tp2, posf)
    input_emb = out5.transpose(2, 4, 0, 1, 3).reshape(BATCH, SEQ, HIDDEN)
    pad_masking = mask_i32.reshape(SEQ, BATCH).T.astype(bool)
    return (input_emb, pad_masking)


# speedup vs baseline: 1.7040x; 1.0612x over previous
import functools

import jax
import jax.numpy as jnp
from jax import lax
from jax.experimental import pallas as pl
from jax.experimental.pallas import tpu as pltpu
from jax.experimental.pallas import tpu_sc as plsc

NUM_ITEM = 1000000
HIDDEN = 64
SEQ = 200
BATCH = 4096
PAD_IDX = 3

NC = 2
NS = 16
LANES = 16
NW = NC * NS
N = BATCH * SEQ
ROWS_PW = N // NW
GSZ = 128
BT = BATCH // GSZ
NG = ROWS_PW // GSZ
PADH = 2 * HIDDEN
HT = HIDDEN // 8
OST = GSZ + 1
NBUF = 4
IBUF = 8
RUNROLL = 4
PAD2 = 2 * PAD_IDX


def _emb_body(xf, table, posf, out5, mask_out, pos_v,
              gbufs, obufs, ibufs, mask_bs, gsems, osems, isems):
    wid = lax.axis_index("s") * NC + lax.axis_index("c")
    base = wid * ROWS_PW
    g0 = wid * NG
    pltpu.sync_copy(posf, pos_v)

    def idx_copy(l, p):
        return pltpu.make_async_copy(
            xf.at[pl.ds(base + l * GSZ, GSZ)], ibufs[p], isems[p])

    def gather_copy(p, q):
        return pltpu.make_async_copy(
            table.at[ibufs[p]], gbufs[q], gsems[q])

    def out_copy(l, obuf, mbuf, sem):
        g = g0 + l
        s = g // BT
        bt = g - s * BT
        return (
            pltpu.make_async_copy(
                obuf.at[:, :, pl.ds(0, GSZ)],
                out5.at[s, :, bt, :, :], sem),
            pltpu.make_async_copy(
                mbuf, mask_out.at[pl.ds(base + l * GSZ, GSZ)], sem),
        )

    iota16 = lax.iota(jnp.int32, LANES)
    hvecs = [iota16 + j * LANES for j in range(HIDDEN // LANES)]
    htv = [hv // 8 for hv in hvecs]
    hmv = [hv % 8 for hv in hvecs]
    for p in range(IBUF):
        idx_copy(p, p).start()
    for q in range(NBUF):
        idx_copy(q, q).wait()
        gather_copy(q, q).start()

    def oct_body(qq, carry):
        for q8 in range(IBUF):
            l = qq * IBUF + q8
            q = q8 % NBUF
            gbuf, obuf, mbuf = gbufs[q], obufs[q8 % 2], mask_bs[q8 % 2]
            @pl.when(l >= 2)
            def _():
                for h in out_copy(l - 2, obuf, mbuf, osems[q8 % 2]):
                    h.wait()
            ibuf = ibufs[q8]
            padcnt = jnp.int32(0)
            for i in range(GSZ // LANES):
                v = ibuf[pl.ds(i * LANES, LANES)]
                ispad = v == PAD2
                mv = jnp.where(ispad, 1, 0)
                mbuf[pl.ds(i * LANES, LANES)] = mv
                padcnt = padcnt + plsc.all_reduce_population_count(ispad)[0]
            gather_copy(q8, q).wait()
            s = (g0 + l) // BT
            pvs = [pos_v[pl.ds(s * HIDDEN + j * LANES, LANES)]
                   for j in range(HIDDEN // LANES)]

            def row_body(r0, carry2):
                for u in range(RUNROLL):
                    r = r0 * RUNROLL + u
                    rvec = jnp.broadcast_to(r, (LANES,))
                    for j in range(HIDDEN // LANES):
                        gv = gbuf[r, pl.ds(j * LANES, LANES)]
                        plsc.store_scatter(obuf, [htv[j], hmv[j], rvec],
                                           gv + pvs[j])
                return carry2

            lax.fori_loop(0, GSZ // RUNROLL, row_body, 0, unroll=False)

            @pl.when(padcnt > 0)
            def _():
                npvs = [
                    jnp.where(ibuf[pl.ds(i * LANES, LANES)] == PAD2,
                              0.0, 1.0)
                    for i in range(GSZ // LANES)
                ]

                def mask_body(h, carry3):
                    for i in range(GSZ // LANES):
                        ov = obuf[h // 8, h % 8, pl.ds(i * LANES, LANES)]
                        obuf[h // 8, h % 8, pl.ds(i * LANES, LANES)] = (
                            ov * npvs[i])
                    return carry3

                lax.fori_loop(0, HIDDEN, mask_body, 0, unroll=False)
            for h in out_copy(l, obuf, mbuf, osems[q8 % 2]):
                h.start()
            @pl.when(l + NBUF < NG)
            def _():
                idx_copy(l + NBUF, (q8 + NBUF) % IBUF).wait()
                gather_copy((q8 + NBUF) % IBUF, q).start()
            @pl.when(l + IBUF < NG)
            def _():
                idx_copy(l + IBUF, q8).start()
        return carry

    lax.fori_loop(0, NG // IBUF, oct_body, 0, unroll=False)
    for h in out_copy(NG - 2, obufs[0], mask_bs[0], osems[0]):
        h.wait()
    for h in out_copy(NG - 1, obufs[1], mask_bs[1], osems[1]):
        h.wait()


_emb_call = pl.kernel(
    _emb_body,
    out_type=[
        jax.ShapeDtypeStruct((SEQ, HT, BT, 8, GSZ), jnp.float32),
        jax.ShapeDtypeStruct((N,), jnp.int32),
    ],
    mesh=plsc.VectorSubcoreMesh(
        core_axis_name="c", subcore_axis_name="s", num_cores=NC,
        num_subcores=NS),
    scratch_types=[
        pltpu.VMEM((SEQ * HIDDEN + LANES,), jnp.float32),
        [pltpu.VMEM((GSZ, HIDDEN), jnp.float32)] * NBUF,
        [pltpu.VMEM((HT, 8, OST), jnp.float32)] * 2,
        [pltpu.VMEM((GSZ,), jnp.int32)] * IBUF,
        [pltpu.VMEM((GSZ,), jnp.int32)] * 2,
        [pltpu.SemaphoreType.DMA] * NBUF,
        [pltpu.SemaphoreType.DMA] * 2,
        [pltpu.SemaphoreType.DMA] * IBUF,
    ],
    compiler_params=pltpu.CompilerParams(
        needs_layout_passes=False, use_tc_tiling_on_sc=False),
)


def kernel(x, item_table, pos_table):
    xf = x.T.reshape(N) * 2
    tp = jnp.pad(item_table, ((0, 0), (0, PADH - HIDDEN)))
    tp2 = tp.reshape(2 * NUM_ITEM, HIDDEN)
    posf = jnp.pad(pos_table.reshape(SEQ * HIDDEN), (0, LANES))
    out5, mask_i32 = _emb_call(xf, tp2, posf)
    input_emb = out5.transpose(2, 4, 0, 1, 3).reshape(BATCH, SEQ, HIDDEN)
    pad_masking = mask_i32.reshape(SEQ, BATCH).T.astype(bool)
    return (input_emb, pad_masking)
